# baseline (speedup 1.0000x reference)
import functools

import jax
import jax.numpy as jnp
from jax import lax
from jax.experimental import pallas as pl
from jax.experimental.pallas import tpu as pltpu
from jax.experimental.pallas import tpu_sc as plsc

D = 64
B = 16384 * 26
NC, NS = 2, 16
NW = NC * NS
SUB = 128
R = B // SUB
R_PER_W = R // NW
K = 8
N_CHUNK = R_PER_W // K

_mesh = plsc.VectorSubcoreMesh(core_axis_name="c", subcore_axis_name="s")


@functools.partial(
    pl.kernel,
    mesh=_mesh,
    compiler_params=pltpu.CompilerParams(use_tc_tiling_on_sc=False),
    out_type=jax.ShapeDtypeStruct((R, SUB, D), jnp.float32),
    scratch_types=[
        pltpu.VMEM((K, SUB), jnp.int32),
        pltpu.VMEM((K, SUB, D), jnp.float32),
        pltpu.SemaphoreType.DMA,
    ],
)
def _gather_kernel(idx_hbm, table_hbm, out_hbm, idx_v, rows_v, sem):
    wid = lax.axis_index("s") * NC + lax.axis_index("c")
    base_w = wid * R_PER_W

    def chunk_body(i, carry):
        base = base_w + i * K
        pltpu.sync_copy(idx_hbm.at[pl.ds(base, K)], idx_v)
        copies = []
        for j in range(K):
            copies.append(
                pltpu.async_copy(table_hbm.at[idx_v.at[j]], rows_v.at[j], sem)
            )
        for c in copies:
            c.wait()
        pltpu.sync_copy(rows_v, out_hbm.at[pl.ds(base, K)])
        return carry

    lax.fori_loop(0, N_CHUNK, chunk_body, 0)


def kernel(context, table):
    idx2 = context.reshape(R, SUB)
    out = _gather_kernel(idx2, table)
    return out.reshape(context.shape[0], context.shape[1], D)


# speedup vs baseline: 1.0159x; 1.0159x over previous
import functools

import jax
import jax.numpy as jnp
from jax import lax
from jax.experimental import pallas as pl
from jax.experimental.pallas import tpu as pltpu
from jax.experimental.pallas import tpu_sc as plsc

D = 64
B = 16384 * 26
NC, NS = 2, 16
NW = NC * NS
SUB = 128
R = B // SUB
R_PER_W = R // NW
NSTR = 4
RPR = 2 * NSTR
NR = R_PER_W // RPR

_mesh = plsc.VectorSubcoreMesh(core_axis_name="c", subcore_axis_name="s")


@functools.partial(
    pl.kernel,
    mesh=_mesh,
    compiler_params=pltpu.CompilerParams(use_tc_tiling_on_sc=False),
    out_type=jax.ShapeDtypeStruct((R, SUB, D), jnp.float32),
    scratch_types=[
        pltpu.VMEM((R_PER_W, SUB), jnp.int32),
        pltpu.VMEM((NSTR, SUB, D), jnp.float32),
        pltpu.VMEM((NSTR, SUB, D), jnp.float32),
        pltpu.SemaphoreType.DMA,
        pltpu.SemaphoreType.DMA,
    ],
)
def _gather_kernel(idx_hbm, table_hbm, out_hbm, idx_v, buf_a, buf_b, sem_a, sem_b):
    wid = lax.axis_index("s") * NC + lax.axis_index("c")
    base = wid * R_PER_W

    pltpu.sync_copy(idx_hbm.at[pl.ds(base, R_PER_W)], idx_v)

    def fire(buf, sem, row0):
        for j in range(NSTR):
            pltpu.async_copy(table_hbm.at[idx_v.at[row0 + j]], buf.at[j], sem)

    def drain(buf, sem):
        for j in range(NSTR):
            pltpu.make_async_copy(table_hbm.at[pl.ds(0, SUB)], buf.at[j], sem).wait()

    fire(buf_a, sem_a, 0)

    def round_body(r, carry):
        row_a = r * RPR
        row_b = row_a + NSTR
        fire(buf_b, sem_b, row_b)
        drain(buf_a, sem_a)
        pltpu.sync_copy(buf_a, out_hbm.at[pl.ds(base + row_a, NSTR)])

        @pl.when(r < NR - 1)
        def _():
            fire(buf_a, sem_a, row_a + RPR)

        drain(buf_b, sem_b)
        pltpu.sync_copy(buf_b, out_hbm.at[pl.ds(base + row_b, NSTR)])
        return carry

    lax.fori_loop(0, NR, round_body, 0)


def kernel(context, table):
    idx2 = context.reshape(R, SUB)
    out = _gather_kernel(idx2, table)
    return out.reshape(context.shape[0], context.shape[1], D)
